# baseline (speedup 1.0000x reference)
import functools

import jax
import jax.numpy as jnp
from jax import lax
from jax.experimental import pallas as pl
from jax.experimental.pallas import tpu as pltpu
from jax.experimental.pallas import tpu_sc as plsc

TW = 7
TH = 7
B, W, H, C = 2, 50, 50, 256
R = 300
NREG = B * R
NC, NS = 2, 16
NWORK = NC * NS
RPAD = 640
SW = W + 1
SB = 2608
SROWS = B * SB
NIV = (TW + TH) * 16


def _tc_body(f_ref, x_ref, y_ref, w_ref, h_ref, b_ref,
             s_ref, idx_ref, inv_ref, p_ref):
    row = lax.broadcasted_iota(jnp.int32, (W + 1, W), 0)
    col = lax.broadcasted_iota(jnp.int32, (W + 1, W), 1)
    tri = (col < row).astype(jnp.float32)
    for b in range(B):
        for h in range(H):
            blk = f_ref[b, :, h, :]
            p_ref[pl.ds(b * SW, SW), h, :] = jnp.dot(
                tri, blk, preferred_element_type=jnp.float32,
                precision=lax.Precision.HIGHEST)
    for b in range(B):
        for u in range(SW):
            blk = p_ref[b * SW + u]
            s_ref[pl.ds(b * SB + u * SW, SW), :] = jnp.dot(
                tri, blk, preferred_element_type=jnp.float32,
                precision=lax.Precision.HIGHEST)
    _idx_math(x_ref, y_ref, w_ref, h_ref, b_ref, idx_ref, inv_ref)


def _idx_math(x_ref, y_ref, w_ref, h_ref, b_ref, idx_ref, inv_ref):
    xi = x_ref[0]
    yi = y_ref[0]
    wi = w_ref[0]
    hi = h_ref[0]
    w_str = lax.shift_right_arithmetic(wi, 1)
    h_str = lax.shift_right_arithmetic(hi, 1)
    l = lax.shift_right_arithmetic(xi - w_str, 4)
    r = lax.shift_right_arithmetic(xi + w_str, 4)
    t = lax.shift_right_arithmetic(yi - h_str, 4)
    b = lax.shift_right_arithmetic(yi + h_str, 4)
    l = jnp.clip(l, 0, W - 1)
    r = jnp.clip(r, 0, W)
    t = jnp.clip(t, 0, H - 1)
    b = jnp.clip(b, 0, H)
    r = jnp.maximum(r, l + 1)
    b = jnp.maximum(b, t + 1)
    w_step = (r - l) // TW
    h_step = (b - t) // TH
    k = lax.broadcasted_iota(jnp.int32, (8, RPAD), 0)
    wb = jnp.where(k == TW, r[None, :], l[None, :] + k * w_step[None, :])
    hb = jnp.where(k == TH, b[None, :], t[None, :] + k * h_step[None, :])
    base = b_ref[0] * SB
    idx_ref[...] = (base[None, None, :] + wb[:, None, :] * SW + hb[None, :, :])
    cw = (wb[1:8] - wb[0:7]).astype(jnp.float32)
    ch = (hb[1:8] - hb[0:7]).astype(jnp.float32)
    inv_ref[...] = 1.0 / jnp.maximum(ch[:, None, :] * cw[None, :, :], 1.0)


def _tc_call(f4, xr, yr, wr, hr, br):
    return pl.pallas_call(
        _tc_body,
        grid=(1,),
        in_specs=[pl.BlockSpec((B, W, H, C), lambda i: (0, 0, 0, 0))]
        + [pl.BlockSpec((1, RPAD), lambda i: (0, 0))] * 5,
        out_specs=[pl.BlockSpec((SROWS, C), lambda i: (0, 0)),
                   pl.BlockSpec((8, 8, RPAD), lambda i: (0, 0, 0)),
                   pl.BlockSpec((TH, TW, RPAD), lambda i: (0, 0, 0))],
        out_shape=[jax.ShapeDtypeStruct((SROWS, C), jnp.float32),
                   jax.ShapeDtypeStruct((8, 8, RPAD), jnp.int32),
                   jax.ShapeDtypeStruct((TH, TW, RPAD), jnp.float32)],
        scratch_shapes=[pltpu.VMEM((B * SW, H, C), jnp.float32)],
    )(f4, xr, yr, wr, hr, br)


def _sc_pool(s_hbm, idx_hbm, out_hbm, idx_v,
             rows_a, rows_b, o_a, o_b,
             gsem_a, gsem_b, osem_a, osem_b):
    wid = lax.axis_index("s") * NC + lax.axis_index("c")
    base = wid * 20
    pltpu.sync_copy(idx_hbm.at[pl.ds(base * 64, 20 * 64)], idx_v)

    rows_bufs = [rows_a, rows_b]
    gsems = [gsem_a, gsem_b]
    osems = [osem_a, osem_b]
    obufs = [o_a, o_b]

    def gather(t, rows, sem):
        return pltpu.async_copy(s_hbm.at[idx_v.at[pl.ds(t * 128, 128)]],
                                rows, sem)

    def gather_wait(t, rows, sem):
        pltpu.make_async_copy(s_hbm.at[idx_v.at[pl.ds(t * 128, 128)]],
                              rows, sem).wait()

    gather(0, rows_a, gsem_a)

    def compute(rows, o_v):
        def vbody(v, _):
            sl = pl.ds(v * 16, 16)
            for sub in range(2):
                ro = sub * 64
                old = [rows[ro + k * 8, sl] for k in range(8)]
                for i in range(TH):
                    new = [rows[ro + k * 8 + i + 1, sl] for k in range(8)]
                    e = [new[k] - old[k] for k in range(8)]
                    for j in range(TW):
                        o_v[sub, i, j, sl] = e[j + 1] - e[j]
                    old = new
            return 0

        lax.fori_loop(0, C // 16, vbody, 0)

    def body(t, _):
        for par in range(2):
            g = 2 * t + par
            rows, gsem = rows_bufs[par], gsems[par]
            o_v, osem = obufs[par], osems[par]

            @pl.when(g < 10)
            def _():
                gather_wait(g, rows, gsem)

                @pl.when(g + 1 < 10)
                def _():
                    gather(g + 1, rows_bufs[(par + 1) % 2],
                           gsems[(par + 1) % 2])

                @pl.when(g >= 2)
                def _():
                    pltpu.make_async_copy(o_v,
                                          out_hbm.at[pl.ds(base + g * 2, 2)],
                                          osem).wait()

                compute(rows, o_v)
                pltpu.async_copy(o_v, out_hbm.at[pl.ds(base + g * 2, 2)],
                                 osem)
        return 0

    lax.fori_loop(0, 5, body, 0)
    pltpu.make_async_copy(o_a, out_hbm.at[pl.ds(base, 2)], osem_a).wait()
    pltpu.make_async_copy(o_b, out_hbm.at[pl.ds(base, 2)], osem_b).wait()


def _sc_call(s_flat, idx_f):
    mesh = plsc.VectorSubcoreMesh(core_axis_name="c", subcore_axis_name="s")
    return pl.kernel(
        _sc_pool,
        out_type=jax.ShapeDtypeStruct((RPAD, TH, TW, C), jnp.float32),
        mesh=mesh,
        scratch_types=[
            pltpu.VMEM((20 * 64,), jnp.int32),
            pltpu.VMEM((128, C), jnp.float32),
            pltpu.VMEM((128, C), jnp.float32),
            pltpu.VMEM((2, TH, TW, C), jnp.float32),
            pltpu.VMEM((2, TH, TW, C), jnp.float32),
            pltpu.SemaphoreType.DMA,
            pltpu.SemaphoreType.DMA,
            pltpu.SemaphoreType.DMA,
            pltpu.SemaphoreType.DMA,
        ],
    )(s_flat, idx_f)


def kernel(feature_maps, regions):
    rflat = regions.reshape(NREG, 4)
    rpad = jnp.pad(rflat, ((0, RPAD - NREG), (0, 0)))
    xr = rpad[:, 0].reshape(1, RPAD)
    yr = rpad[:, 1].reshape(1, RPAD)
    wr = rpad[:, 2].reshape(1, RPAD)
    hr = rpad[:, 3].reshape(1, RPAD)
    br = jnp.minimum(jnp.arange(RPAD, dtype=jnp.int32) // R,
                     B - 1).reshape(1, RPAD)
    s_flat, idx3, inv3 = _tc_call(feature_maps, xr, yr, wr, hr, br)
    idx_f = idx3.transpose(2, 0, 1).reshape(RPAD * 64)
    invc = inv3.transpose(2, 0, 1)[:NREG].reshape(B, R, TH, TW, 1)

    sums = _sc_call(s_flat, idx_f)
    return sums[:NREG].reshape(B, R, TH, TW, C) * invc


# speedup vs baseline: 1.7556x; 1.7556x over previous
import functools

import jax
import jax.numpy as jnp
from jax import lax
from jax.experimental import pallas as pl
from jax.experimental.pallas import tpu as pltpu
from jax.experimental.pallas import tpu_sc as plsc

TW = 7
TH = 7
B, W, H, C = 2, 50, 50, 256
R = 300
NREG = B * R
NC, NS = 2, 16
NWORK = NC * NS
RPAD = 608
SW = W + 1
SB = 2608
SROWS = B * SB
NIV = (TW + TH) * 16


def _tc_body(f_ref, x_ref, y_ref, w_ref, h_ref, b_ref,
             s_ref, idx_ref, inv_ref, p_ref):
    row = lax.broadcasted_iota(jnp.int32, (W + 1, W), 0)
    col = lax.broadcasted_iota(jnp.int32, (W + 1, W), 1)
    tri = (col < row).astype(jnp.float32)
    for b in range(B):
        for h in range(H):
            blk = f_ref[b, :, h, :]
            p_ref[pl.ds(b * SW, SW), h, :] = jnp.dot(
                tri, blk, preferred_element_type=jnp.float32,
                precision=lax.Precision.HIGHEST)
    for b in range(B):
        for u in range(SW):
            blk = p_ref[b * SW + u]
            s_ref[pl.ds(b * SB + u * SW, SW), :] = jnp.dot(
                tri, blk, preferred_element_type=jnp.float32,
                precision=lax.Precision.HIGHEST)
    _idx_math(x_ref, y_ref, w_ref, h_ref, b_ref, idx_ref, inv_ref)


def _idx_math(x_ref, y_ref, w_ref, h_ref, b_ref, idx_ref, inv_ref):
    xi = x_ref[0]
    yi = y_ref[0]
    wi = w_ref[0]
    hi = h_ref[0]
    w_str = lax.shift_right_arithmetic(wi, 1)
    h_str = lax.shift_right_arithmetic(hi, 1)
    l = lax.shift_right_arithmetic(xi - w_str, 4)
    r = lax.shift_right_arithmetic(xi + w_str, 4)
    t = lax.shift_right_arithmetic(yi - h_str, 4)
    b = lax.shift_right_arithmetic(yi + h_str, 4)
    l = jnp.clip(l, 0, W - 1)
    r = jnp.clip(r, 0, W)
    t = jnp.clip(t, 0, H - 1)
    b = jnp.clip(b, 0, H)
    r = jnp.maximum(r, l + 1)
    b = jnp.maximum(b, t + 1)
    w_step = (r - l) // TW
    h_step = (b - t) // TH
    k = lax.broadcasted_iota(jnp.int32, (8, RPAD), 0)
    wb = jnp.where(k == TW, r[None, :], l[None, :] + k * w_step[None, :])
    hb = jnp.where(k == TH, b[None, :], t[None, :] + k * h_step[None, :])
    base = b_ref[0] * SB
    idx_ref[...] = (base[None, None, :] + wb[:, None, :] * SW + hb[None, :, :])
    cw = (wb[1:8] - wb[0:7]).astype(jnp.float32)
    ch = (hb[1:8] - hb[0:7]).astype(jnp.float32)
    inv_ref[...] = 1.0 / jnp.maximum(ch[:, None, :] * cw[None, :, :], 1.0)


def _tc_call(f4, xr, yr, wr, hr, br):
    return pl.pallas_call(
        _tc_body,
        grid=(1,),
        in_specs=[pl.BlockSpec((B, W, H, C), lambda i: (0, 0, 0, 0))]
        + [pl.BlockSpec((1, RPAD), lambda i: (0, 0))] * 5,
        out_specs=[pl.BlockSpec((SROWS, C), lambda i: (0, 0)),
                   pl.BlockSpec((8, 8, RPAD), lambda i: (0, 0, 0)),
                   pl.BlockSpec((TH, TW, RPAD), lambda i: (0, 0, 0))],
        out_shape=[jax.ShapeDtypeStruct((SROWS, C), jnp.float32),
                   jax.ShapeDtypeStruct((8, 8, RPAD), jnp.int32),
                   jax.ShapeDtypeStruct((TH, TW, RPAD), jnp.float32)],
        scratch_shapes=[pltpu.VMEM((B * SW, H, C), jnp.float32)],
    )(f4, xr, yr, wr, hr, br)


def _sc_pool(s_hbm, idx_hbm, out_hbm, idx_v,
             rows_a, rows_b, rows_c, rows_d, o_a, o_b,
             gsem_a, gsem_b, gsem_c, gsem_d, osem_a, osem_b):
    wid = lax.axis_index("s") * NC + lax.axis_index("c")
    cnt = jnp.where(wid < 24, 19, 18)
    base = wid * 18 + jnp.minimum(wid, 24)
    pltpu.sync_copy(idx_hbm.at[pl.ds(base * 64, 19 * 64)], idx_v)

    rows_bufs = [rows_a, rows_b, rows_c, rows_d]
    gsems = [gsem_a, gsem_b, gsem_c, gsem_d]

    def gather(g, rows, sem):
        return pltpu.async_copy(s_hbm.at[idx_v.at[pl.ds(g * 64, 64)]],
                                rows, sem)

    def gather_wait(g, rows, sem):
        pltpu.make_async_copy(s_hbm.at[idx_v.at[pl.ds(g * 64, 64)]],
                              rows, sem).wait()

    for p in range(3):
        @pl.when(p < cnt)
        def _(p=p):
            gather(p, rows_bufs[p], gsems[p])

    def compute(g, rows, o_v):
        def vbody(v, _):
            sl = pl.ds(v * 16, 16)
            old = [rows[k * 8, sl] for k in range(8)]
            for i in range(TH):
                new = [rows[k * 8 + i + 1, sl] for k in range(8)]
                e = [new[k] - old[k] for k in range(8)]
                for j in range(TW):
                    o_v[i, j, sl] = e[j + 1] - e[j]
                old = new
            return 0

        lax.fori_loop(0, C // 16, vbody, 0)

    def body(t, _):
        for par in range(4):
            g = 4 * t + par
            rows, gsem = rows_bufs[par], gsems[par]
            o_v, osem = (o_a, osem_a) if par % 2 == 0 else (o_b, osem_b)

            @pl.when(g < cnt)
            def _():
                gather_wait(g, rows, gsem)

                @pl.when(g + 3 < cnt)
                def _():
                    gather(g + 3, rows_bufs[(par + 3) % 4],
                           gsems[(par + 3) % 4])

                @pl.when(g >= 2)
                def _():
                    pltpu.make_async_copy(o_v, out_hbm.at[base + g],
                                          osem).wait()

                compute(g, rows, o_v)
                pltpu.async_copy(o_v, out_hbm.at[base + g], osem)
        return 0

    lax.fori_loop(0, 5, body, 0)
    pltpu.make_async_copy(o_a, out_hbm.at[base], osem_a).wait()
    pltpu.make_async_copy(o_b, out_hbm.at[base], osem_b).wait()


def _sc_call(s_flat, idx_f):
    mesh = plsc.VectorSubcoreMesh(core_axis_name="c", subcore_axis_name="s")
    return pl.kernel(
        _sc_pool,
        out_type=jax.ShapeDtypeStruct((NREG, TH, TW, C), jnp.float32),
        mesh=mesh,
        scratch_types=[
            pltpu.VMEM((19 * 64,), jnp.int32),
            pltpu.VMEM((64, C), jnp.float32),
            pltpu.VMEM((64, C), jnp.float32),
            pltpu.VMEM((64, C), jnp.float32),
            pltpu.VMEM((64, C), jnp.float32),
            pltpu.VMEM((TH, TW, C), jnp.float32),
            pltpu.VMEM((TH, TW, C), jnp.float32),
            pltpu.SemaphoreType.DMA,
            pltpu.SemaphoreType.DMA,
            pltpu.SemaphoreType.DMA,
            pltpu.SemaphoreType.DMA,
            pltpu.SemaphoreType.DMA,
            pltpu.SemaphoreType.DMA,
        ],
    )(s_flat, idx_f)


def kernel(feature_maps, regions):
    rflat = regions.reshape(NREG, 4)
    rpad = jnp.pad(rflat, ((0, RPAD - NREG), (0, 0)))
    xr = rpad[:, 0].reshape(1, RPAD)
    yr = rpad[:, 1].reshape(1, RPAD)
    wr = rpad[:, 2].reshape(1, RPAD)
    hr = rpad[:, 3].reshape(1, RPAD)
    br = jnp.minimum(jnp.arange(RPAD, dtype=jnp.int32) // R,
                     B - 1).reshape(1, RPAD)
    s_flat, idx3, inv3 = _tc_call(feature_maps, xr, yr, wr, hr, br)
    idx_f = idx3.transpose(2, 0, 1).reshape(RPAD * 64)
    invc = inv3.transpose(2, 0, 1)[:NREG].reshape(B, R, TH, TW, 1)

    sums = _sc_call(s_flat, idx_f)
    return sums.reshape(B, R, TH, TW, C) * invc


# speedup vs baseline: 1.7686x; 1.0074x over previous
import functools

import jax
import jax.numpy as jnp
from jax import lax
from jax.experimental import pallas as pl
from jax.experimental.pallas import tpu as pltpu
from jax.experimental.pallas import tpu_sc as plsc

TW = 7
TH = 7
B, W, H, C = 2, 50, 50, 256
R = 300
NREG = B * R
NC, NS = 2, 16
NWORK = NC * NS
RPAD = 608
SW = W + 1
SB = 2608
SROWS = B * SB
NIV = (TW + TH) * 16


def _tc_body(f_ref, x_ref, y_ref, w_ref, h_ref, b_ref,
             s_ref, idx_ref, inv_ref, p_ref):
    row = lax.broadcasted_iota(jnp.int32, (W + 1, W), 0)
    col = lax.broadcasted_iota(jnp.int32, (W + 1, W), 1)
    tri = (col < row).astype(jnp.float32)
    for b in range(B):
        for h in range(H):
            blk = f_ref[b, :, h, :]
            p_ref[pl.ds(b * SW, SW), h, :] = jnp.dot(
                tri, blk, preferred_element_type=jnp.float32,
                precision=lax.Precision.HIGHEST)
    for b in range(B):
        for u in range(SW):
            blk = p_ref[b * SW + u]
            s_ref[pl.ds(b * SB + u * SW, SW), :] = jnp.dot(
                tri, blk, preferred_element_type=jnp.float32,
                precision=lax.Precision.HIGHEST)
    _idx_math(x_ref, y_ref, w_ref, h_ref, b_ref, idx_ref, inv_ref)


def _idx_math(x_ref, y_ref, w_ref, h_ref, b_ref, idx_ref, inv_ref):
    xi = x_ref[0]
    yi = y_ref[0]
    wi = w_ref[0]
    hi = h_ref[0]
    w_str = lax.shift_right_arithmetic(wi, 1)
    h_str = lax.shift_right_arithmetic(hi, 1)
    l = lax.shift_right_arithmetic(xi - w_str, 4)
    r = lax.shift_right_arithmetic(xi + w_str, 4)
    t = lax.shift_right_arithmetic(yi - h_str, 4)
    b = lax.shift_right_arithmetic(yi + h_str, 4)
    l = jnp.clip(l, 0, W - 1)
    r = jnp.clip(r, 0, W)
    t = jnp.clip(t, 0, H - 1)
    b = jnp.clip(b, 0, H)
    r = jnp.maximum(r, l + 1)
    b = jnp.maximum(b, t + 1)
    w_step = (r - l) // TW
    h_step = (b - t) // TH
    k = lax.broadcasted_iota(jnp.int32, (8, RPAD), 0)
    wb = jnp.where(k == TW, r[None, :], l[None, :] + k * w_step[None, :])
    hb = jnp.where(k == TH, b[None, :], t[None, :] + k * h_step[None, :])
    base = b_ref[0] * SB
    idx_ref[...] = (base[None, None, :] + wb[:, None, :] * SW + hb[None, :, :])
    cw = (wb[1:8] - wb[0:7]).astype(jnp.float32)
    ch = (hb[1:8] - hb[0:7]).astype(jnp.float32)
    inv_ref[...] = 1.0 / jnp.maximum(ch[:, None, :] * cw[None, :, :], 1.0)


def _tc_call(f4, xr, yr, wr, hr, br):
    return pl.pallas_call(
        _tc_body,
        grid=(1,),
        in_specs=[pl.BlockSpec((B, W, H, C), lambda i: (0, 0, 0, 0))]
        + [pl.BlockSpec((1, RPAD), lambda i: (0, 0))] * 5,
        out_specs=[pl.BlockSpec((SROWS, C), lambda i: (0, 0)),
                   pl.BlockSpec((8, 8, RPAD), lambda i: (0, 0, 0)),
                   pl.BlockSpec((TH, TW, RPAD), lambda i: (0, 0, 0))],
        out_shape=[jax.ShapeDtypeStruct((SROWS, C), jnp.float32),
                   jax.ShapeDtypeStruct((8, 8, RPAD), jnp.int32),
                   jax.ShapeDtypeStruct((TH, TW, RPAD), jnp.float32)],
        scratch_shapes=[pltpu.VMEM((B * SW, H, C), jnp.float32)],
    )(f4, xr, yr, wr, hr, br)


def _sc_pool(s_hbm, idx_hbm, out_hbm, idx_v,
             rows_a, rows_b, rows_c, rows_d, rows_e, o_a, o_b,
             gsem_a, gsem_b, gsem_c, gsem_d, gsem_e, osem_a, osem_b):
    wid = lax.axis_index("s") * NC + lax.axis_index("c")
    cnt = jnp.where(wid < 24, 19, 18)
    base = wid * 18 + jnp.minimum(wid, 24)
    pltpu.sync_copy(idx_hbm.at[pl.ds(base * 64, 19 * 64)], idx_v)

    rows_bufs = [rows_a, rows_b, rows_c, rows_d, rows_e]
    gsems = [gsem_a, gsem_b, gsem_c, gsem_d, gsem_e]

    def gather(g, rows, sem):
        return pltpu.async_copy(s_hbm.at[idx_v.at[pl.ds(g * 64, 64)]],
                                rows, sem)

    def gather_wait(g, rows, sem):
        pltpu.make_async_copy(s_hbm.at[idx_v.at[pl.ds(g * 64, 64)]],
                              rows, sem).wait()

    for p in range(4):
        @pl.when(p < cnt)
        def _(p=p):
            gather(p, rows_bufs[p], gsems[p])

    def compute(g, rows, o_v):
        def vbody(v, _):
            sl = pl.ds(v * 16, 16)
            old = [rows[k * 8, sl] for k in range(8)]
            for i in range(TH):
                new = [rows[k * 8 + i + 1, sl] for k in range(8)]
                e = [new[k] - old[k] for k in range(8)]
                for j in range(TW):
                    o_v[i, j, sl] = e[j + 1] - e[j]
                old = new
            return 0

        lax.fori_loop(0, C // 16, vbody, 0)

    def body(t, _):
        for par in range(5):
            g = 5 * t + par
            rows, gsem = rows_bufs[par], gsems[par]
            o_v, osem = (o_a, osem_a) if par % 2 == 0 else (o_b, osem_b)

            @pl.when(g < cnt)
            def _():
                gather_wait(g, rows, gsem)

                @pl.when(g + 4 < cnt)
                def _():
                    gather(g + 4, rows_bufs[(par + 4) % 5],
                           gsems[(par + 4) % 5])

                @pl.when(g >= 2)
                def _():
                    pltpu.make_async_copy(o_v, out_hbm.at[base + g],
                                          osem).wait()

                compute(g, rows, o_v)
                pltpu.async_copy(o_v, out_hbm.at[base + g], osem)
        return 0

    lax.fori_loop(0, 4, body, 0)
    pltpu.make_async_copy(o_a, out_hbm.at[base], osem_a).wait()
    pltpu.make_async_copy(o_b, out_hbm.at[base], osem_b).wait()


def _sc_call(s_flat, idx_f):
    mesh = plsc.VectorSubcoreMesh(core_axis_name="c", subcore_axis_name="s")
    return pl.kernel(
        _sc_pool,
        out_type=jax.ShapeDtypeStruct((NREG, TH, TW, C), jnp.float32),
        mesh=mesh,
        scratch_types=[
            pltpu.VMEM((19 * 64,), jnp.int32),
            pltpu.VMEM((64, C), jnp.float32),
            pltpu.VMEM((64, C), jnp.float32),
            pltpu.VMEM((64, C), jnp.float32),
            pltpu.VMEM((64, C), jnp.float32),
            pltpu.VMEM((64, C), jnp.float32),
            pltpu.VMEM((TH, TW, C), jnp.float32),
            pltpu.VMEM((TH, TW, C), jnp.float32),
            pltpu.SemaphoreType.DMA,
            pltpu.SemaphoreType.DMA,
            pltpu.SemaphoreType.DMA,
            pltpu.SemaphoreType.DMA,
            pltpu.SemaphoreType.DMA,
            pltpu.SemaphoreType.DMA,
            pltpu.SemaphoreType.DMA,
        ],
    )(s_flat, idx_f)


def kernel(feature_maps, regions):
    rflat = regions.reshape(NREG, 4)
    rpad = jnp.pad(rflat, ((0, RPAD - NREG), (0, 0)))
    xr = rpad[:, 0].reshape(1, RPAD)
    yr = rpad[:, 1].reshape(1, RPAD)
    wr = rpad[:, 2].reshape(1, RPAD)
    hr = rpad[:, 3].reshape(1, RPAD)
    br = jnp.minimum(jnp.arange(RPAD, dtype=jnp.int32) // R,
                     B - 1).reshape(1, RPAD)
    s_flat, idx3, inv3 = _tc_call(feature_maps, xr, yr, wr, hr, br)
    idx_f = idx3.transpose(2, 0, 1).reshape(RPAD * 64)
    invc = inv3.transpose(2, 0, 1)[:NREG].reshape(B, R, TH, TW, 1)

    sums = _sc_call(s_flat, idx_f)
    return sums.reshape(B, R, TH, TW, C) * invc
